# baseline (speedup 1.0000x reference)
import functools

import jax
import jax.numpy as jnp
from jax import lax
from jax.experimental import pallas as pl
from jax.experimental.pallas import tpu as pltpu
from jax.experimental.pallas import tpu_sc as plsc

N_NODES = 10000
N_EDGES = 320000
D_IN = 128
D_HID = 64
D_OUT = 128

NC = 2
NS = 16
NW = NC * NS
CHUNK = 64
EPT = 10240
E_PAD = EPT * NW
NCHUNK = EPT // CHUNK
NBUF = 4
NQUAD = NCHUNK // NBUF
SLAB = 640
N_PAD = 10240


def _sc_aggregate(x, srcp, dst3):
  mesh = plsc.VectorSubcoreMesh(core_axis_name="c", subcore_axis_name="s")

  @functools.partial(
      pl.kernel,
      mesh=mesh,
      out_type=jax.ShapeDtypeStruct((NC, N_NODES, D_IN), jnp.float32),
      scratch_types=[
          pltpu.VMEM((NCHUNK // 2, 2 * CHUNK), jnp.int32),
          pltpu.VMEM((1, CHUNK), jnp.int32),
          pltpu.VMEM((1, CHUNK), jnp.int32),
          pltpu.VMEM((1, CHUNK), jnp.int32),
          pltpu.VMEM((1, CHUNK), jnp.int32),
          pltpu.VMEM((CHUNK, D_IN), jnp.float32),
          pltpu.VMEM((CHUNK, D_IN), jnp.float32),
          pltpu.VMEM((CHUNK, D_IN), jnp.float32),
          pltpu.VMEM((CHUNK, D_IN), jnp.float32),
          pltpu.VMEM_SHARED((N_PAD, D_IN), jnp.float32),
          pltpu.SemaphoreType.DMA,
          pltpu.SemaphoreType.DMA,
          pltpu.SemaphoreType.DMA,
          pltpu.SemaphoreType.DMA,
          pltpu.SemaphoreType.DMA,
          pltpu.SemaphoreType.DMA,
          pltpu.SemaphoreType.DMA,
          pltpu.SemaphoreType.DMA,
          pltpu.SemaphoreType.DMA,
          pltpu.SemaphoreType.DMA,
          pltpu.SemaphoreType.DMA,
          pltpu.SemaphoreType.DMA,
      ],
  )
  def agg(x_hbm, src_hbm, dst_hbm, out_hbm, src_v, sd0, sd1, sd2, sd3,
          r0, r1, r2, r3, acc_sh, ssd0, ssd1, ssd2, ssd3,
          ssa0, ssa1, ssa2, ssa3, ssc0, ssc1, ssc2, ssc3):
    sd = (sd0, sd1, sd2, sd3)
    rows = (r0, r1, r2, r3)
    ssd = (ssd0, ssd1, ssd2, ssd3)
    ssa = (ssa0, ssa1, ssa2, ssa3)
    ssc = (ssc0, ssc1, ssc2, ssc3)
    c = lax.axis_index("c")
    s = lax.axis_index("s")
    wid = c * NS + s

    def gather(ci_row, half, b):
      idx = src_v.at[ci_row, pl.ds(half * CHUNK, CHUNK)]
      return pltpu.async_copy(x_hbm.at[idx], rows[b], ssa[b])

    for buf in (r0, r1):
      @pl.loop(0, CHUNK)
      def _(r):
        for j in range(D_IN // 16):
          buf[r, pl.ds(j * 16, 16)] = jnp.zeros((16,), jnp.float32)

    nz = jnp.where(s < NS - 1, SLAB // CHUNK, 6)

    @pl.loop(0, nz)
    def _(j):
      buf = rows[0]
      pltpu.sync_copy(buf, acc_sh.at[pl.ds(s * SLAB + j * CHUNK, CHUNK)])

    @pl.when(s == NS - 1)
    def _():
      pltpu.sync_copy(r1.at[pl.ds(0, 16)],
                      acc_sh.at[pl.ds((NS - 1) * SLAB + 6 * CHUNK, 16)])

    plsc.subcore_barrier()

    pltpu.sync_copy(src_hbm.at[wid], src_v)
    for b in range(2):
      pltpu.async_copy(dst_hbm.at[wid, b], sd[b].at[0], ssd[b])
      gather(b // 2, b % 2, b)

    @pl.loop(0, NQUAD)
    def _(k):
      for b in range(NBUF):
        b2 = (b + 2) % NBUF
        ci = NBUF * k + b
        pltpu.make_async_copy(x_hbm.at[src_v.at[2 * k + b // 2,
                                                pl.ds((b % 2) * CHUNK, CHUNK)]],
                              rows[b], ssa[b]).wait()
        pltpu.make_async_copy(dst_hbm.at[wid, ci], sd[b].at[0], ssd[b]).wait()
        pltpu.sync_copy(rows[b], acc_sh.at[sd[b].at[0]], add=True)

        @pl.when(ci + 2 < NCHUNK)
        def _():
          pltpu.async_copy(dst_hbm.at[wid, ci + 2], sd[b2].at[0], ssd[b2])
          gather(2 * k + (b + 2) // 2, b % 2, b2)

    plsc.subcore_barrier()

    row0 = pl.multiple_of(s * SLAB, 8)

    @pl.when(s < NS - 1)
    def _():
      pltpu.sync_copy(acc_sh.at[pl.ds(row0, SLAB)],
                      out_hbm.at[c, pl.ds(row0, SLAB)])

    last = N_NODES - (NS - 1) * SLAB

    @pl.when(s == NS - 1)
    def _():
      pltpu.sync_copy(acc_sh.at[pl.ds((NS - 1) * SLAB, last)],
                      out_hbm.at[c, pl.ds((NS - 1) * SLAB, last)])

  return agg(x, srcp, dst3)


BLK = 1000


def _mlp(partials, W1, b1, W2, b2):
  def body(p_ref, w1_ref, b1_ref, w2_ref, b2_ref, o_ref):
    ax = p_ref[0] + p_ref[1]
    h = jnp.dot(ax, w1_ref[...], preferred_element_type=jnp.float32)
    h = jnp.maximum(h + b1_ref[...], 0.0)
    o_ref[...] = (jnp.dot(h, w2_ref[...], preferred_element_type=jnp.float32)
                  + b2_ref[...])

  return pl.pallas_call(
      body,
      grid=(N_NODES // BLK,),
      in_specs=[
          pl.BlockSpec((NC, BLK, D_IN), lambda i: (0, i, 0)),
          pl.BlockSpec((D_IN, D_HID), lambda i: (0, 0)),
          pl.BlockSpec((1, D_HID), lambda i: (0, 0)),
          pl.BlockSpec((D_HID, D_OUT), lambda i: (0, 0)),
          pl.BlockSpec((1, D_OUT), lambda i: (0, 0)),
      ],
      out_specs=pl.BlockSpec((BLK, D_OUT), lambda i: (i, 0)),
      out_shape=jax.ShapeDtypeStruct((N_NODES, D_OUT), jnp.float32),
  )(partials, W1, b1.reshape(1, D_HID), W2, b2.reshape(1, D_OUT))


def kernel(x, edge_index, W1, b1, W2, b2):
  ei = edge_index.astype(jnp.int32)
  npad = E_PAD - N_EDGES
  pad_src = jnp.arange(npad, dtype=jnp.int32) % N_NODES
  pad_dst = N_NODES + (jnp.arange(npad, dtype=jnp.int32) % (N_PAD - N_NODES))
  src = jnp.concatenate([ei[0], pad_src])
  dst = jnp.concatenate([ei[1], pad_dst])
  srcp = src.reshape(NW, NCHUNK // 2, 2 * CHUNK)
  dst3 = dst.reshape(NW, NCHUNK, CHUNK)
  partials = _sc_aggregate(x, srcp, dst3)
  return _mlp(partials, W1, b1, W2, b2)


# speedup vs baseline: 1.1851x; 1.1851x over previous
import functools

import jax
import jax.numpy as jnp
from jax import lax
from jax.experimental import pallas as pl
from jax.experimental.pallas import tpu as pltpu
from jax.experimental.pallas import tpu_sc as plsc

N_NODES = 10000
N_EDGES = 320000
D_IN = 128
D_HID = 64
D_OUT = 128

NC = 2
NS = 16
NW = NC * NS
CHUNK = 128
EPT = 10240
E_PAD = EPT * NW
NCHUNK = EPT // CHUNK
NPAIR = NCHUNK // 2
SLAB = 640
ZCH = 80
N_PAD = 10240


def _sc_aggregate(x, src3, dst3):
  mesh = plsc.VectorSubcoreMesh(core_axis_name="c", subcore_axis_name="s")

  @functools.partial(
      pl.kernel,
      mesh=mesh,
      out_type=jax.ShapeDtypeStruct((NC, N_NODES, D_IN), jnp.float32),
      scratch_types=[
          pltpu.VMEM((NCHUNK, CHUNK), jnp.int32),
          pltpu.VMEM((1, CHUNK), jnp.int32),
          pltpu.VMEM((1, CHUNK), jnp.int32),
          pltpu.VMEM((CHUNK, D_IN), jnp.float32),
          pltpu.VMEM((CHUNK, D_IN), jnp.float32),
          pltpu.VMEM_SHARED((N_PAD, D_IN), jnp.float32),
          pltpu.SemaphoreType.DMA,
          pltpu.SemaphoreType.DMA,
          pltpu.SemaphoreType.DMA,
          pltpu.SemaphoreType.DMA,
      ],
  )
  def agg(x_hbm, src_hbm, dst_hbm, out_hbm, src_v, sd0, sd1, rows0, rows1,
          acc_sh, ssd0, ssd1, ssa0, ssa1):
    sd = (sd0, sd1)
    rows = (rows0, rows1)
    ssd = (ssd0, ssd1)
    ssa = (ssa0, ssa1)
    c = lax.axis_index("c")
    s = lax.axis_index("s")
    wid = c * NS + s

    @pl.loop(0, ZCH)
    def _(r):
      for j in range(D_IN // 16):
        rows0[r, pl.ds(j * 16, 16)] = jnp.zeros((16,), jnp.float32)

    nz = jnp.where(s < NS - 1, SLAB // ZCH, (N_NODES - (NS - 1) * SLAB) // ZCH)

    @pl.loop(0, nz)
    def _(j):
      pltpu.sync_copy(rows0.at[pl.ds(0, ZCH)],
                      acc_sh.at[pl.ds(s * SLAB + j * ZCH, ZCH)])

    plsc.subcore_barrier()

    pltpu.sync_copy(src_hbm.at[wid], src_v)
    for b in range(2):
      pltpu.async_copy(dst_hbm.at[wid, b], sd[b].at[0], ssd[b])
      pltpu.async_copy(x_hbm.at[src_v.at[b]], rows[b], ssa[b])

    @pl.loop(0, NPAIR)
    def _(k):
      for b in range(2):
        ci = 2 * k + b
        pltpu.make_async_copy(x_hbm.at[src_v.at[ci]], rows[b], ssa[b]).wait()
        pltpu.make_async_copy(dst_hbm.at[wid, ci], sd[b].at[0], ssd[b]).wait()
        pltpu.sync_copy(rows[b], acc_sh.at[sd[b].at[0]], add=True)

        @pl.when(ci + 2 < NCHUNK)
        def _():
          pltpu.async_copy(dst_hbm.at[wid, ci + 2], sd[b].at[0], ssd[b])
          pltpu.async_copy(x_hbm.at[src_v.at[ci + 2]], rows[b], ssa[b])

    plsc.subcore_barrier()

    row0 = pl.multiple_of(s * SLAB, 8)

    @pl.when(s < NS - 1)
    def _():
      pltpu.sync_copy(acc_sh.at[pl.ds(row0, SLAB)],
                      out_hbm.at[c, pl.ds(row0, SLAB)])

    last = N_NODES - (NS - 1) * SLAB

    @pl.when(s == NS - 1)
    def _():
      pltpu.sync_copy(acc_sh.at[pl.ds((NS - 1) * SLAB, last)],
                      out_hbm.at[c, pl.ds((NS - 1) * SLAB, last)])

  return agg(x, src3, dst3)


BLK = 2000


def _mlp(partials, W1, b1, W2, b2):
  def body(p_ref, w1_ref, b1_ref, w2_ref, b2_ref, o_ref):
    ax = p_ref[0] + p_ref[1]
    h = jnp.dot(ax.astype(jnp.bfloat16), w1_ref[...].astype(jnp.bfloat16),
                preferred_element_type=jnp.float32)
    h = jnp.maximum(h + b1_ref[...], 0.0)
    o_ref[...] = (jnp.dot(h.astype(jnp.bfloat16),
                          w2_ref[...].astype(jnp.bfloat16),
                          preferred_element_type=jnp.float32) + b2_ref[...])

  return pl.pallas_call(
      body,
      grid=(N_NODES // BLK,),
      in_specs=[
          pl.BlockSpec((NC, BLK, D_IN), lambda i: (0, i, 0)),
          pl.BlockSpec((D_IN, D_HID), lambda i: (0, 0)),
          pl.BlockSpec((1, D_HID), lambda i: (0, 0)),
          pl.BlockSpec((D_HID, D_OUT), lambda i: (0, 0)),
          pl.BlockSpec((1, D_OUT), lambda i: (0, 0)),
      ],
      out_specs=pl.BlockSpec((BLK, D_OUT), lambda i: (i, 0)),
      out_shape=jax.ShapeDtypeStruct((N_NODES, D_OUT), jnp.float32),
  )(partials, W1, b1.reshape(1, D_HID), W2, b2.reshape(1, D_OUT))


def kernel(x, edge_index, W1, b1, W2, b2):
  ei = edge_index.astype(jnp.int32)
  npad = E_PAD - N_EDGES
  pad_src = jnp.arange(npad, dtype=jnp.int32) % N_NODES
  src = jnp.concatenate([ei[0], pad_src])
  pad_dst = N_NODES + (jnp.arange(npad, dtype=jnp.int32) % (N_PAD - N_NODES))
  dst = jnp.concatenate([ei[1], pad_dst])
  src3 = src.reshape(NW, NCHUNK, CHUNK)
  dst3 = dst.reshape(NW, NCHUNK, CHUNK)
  partials = _sc_aggregate(x, src3, dst3)
  return _mlp(partials, W1, b1, W2, b2)


# speedup vs baseline: 1.1946x; 1.0080x over previous
import functools

import jax
import jax.numpy as jnp
from jax import lax
from jax.experimental import pallas as pl
from jax.experimental.pallas import tpu as pltpu
from jax.experimental.pallas import tpu_sc as plsc

N_NODES = 10000
N_EDGES = 320000
D_IN = 128
D_HID = 64
D_OUT = 128

NC = 2
NS = 16
NW = NC * NS
CHUNK = 128
EPT = 10240
E_PAD = EPT * NW
NCHUNK = EPT // CHUNK
NPAIR = NCHUNK // 2
SLAB = 640
ZCH = 80
N_PAD = 10240


def _sc_aggregate(x, src3, dst3):
  mesh = plsc.VectorSubcoreMesh(core_axis_name="c", subcore_axis_name="s")

  @functools.partial(
      pl.kernel,
      mesh=mesh,
      out_type=jax.ShapeDtypeStruct((NC, N_NODES, D_IN), jnp.float32),
      scratch_types=[
          pltpu.VMEM((NCHUNK, CHUNK), jnp.int32),
          pltpu.VMEM((1, CHUNK), jnp.int32),
          pltpu.VMEM((1, CHUNK), jnp.int32),
          pltpu.VMEM((CHUNK, D_IN), jnp.float32),
          pltpu.VMEM((CHUNK, D_IN), jnp.float32),
          pltpu.VMEM_SHARED((N_PAD, D_IN), jnp.float32),
          pltpu.SemaphoreType.DMA,
          pltpu.SemaphoreType.DMA,
          pltpu.SemaphoreType.DMA,
          pltpu.SemaphoreType.DMA,
          pltpu.SemaphoreType.DMA,
      ],
  )
  def agg(x_hbm, src_hbm, dst_hbm, out_hbm, src_v, sd0, sd1, rows0, rows1,
          acc_sh, ssd0, ssd1, ssa0, ssa1, ssz):
    sd = (sd0, sd1)
    rows = (rows0, rows1)
    ssd = (ssd0, ssd1)
    ssa = (ssa0, ssa1)
    c = lax.axis_index("c")
    s = lax.axis_index("s")
    wid = c * NS + s

    pltpu.async_copy(src_hbm.at[wid], src_v, ssz)

    @pl.loop(0, ZCH)
    def _(r):
      for j in range(D_IN // 16):
        rows0[r, pl.ds(j * 16, 16)] = jnp.zeros((16,), jnp.float32)

    nz = jnp.where(s < NS - 1, SLAB // ZCH, (N_NODES - (NS - 1) * SLAB) // ZCH)

    @pl.loop(0, nz)
    def _(j):
      pltpu.async_copy(rows0.at[pl.ds(0, ZCH)],
                       acc_sh.at[pl.ds(s * SLAB + j * ZCH, ZCH)], ssd[0])

    @pl.loop(0, nz)
    def _(j):
      pltpu.make_async_copy(rows0.at[pl.ds(0, ZCH)],
                            acc_sh.at[pl.ds(s * SLAB, ZCH)], ssd[0]).wait()

    pltpu.make_async_copy(src_hbm.at[wid], src_v, ssz).wait()

    for b in range(2):
      pltpu.async_copy(dst_hbm.at[wid, b], sd[b].at[0], ssd[b])
      pltpu.async_copy(x_hbm.at[src_v.at[b]], rows[b], ssa[b])

    plsc.subcore_barrier()

    @pl.loop(0, NPAIR)
    def _(k):
      for b in range(2):
        ci = 2 * k + b
        pltpu.make_async_copy(x_hbm.at[src_v.at[ci]], rows[b], ssa[b]).wait()
        pltpu.make_async_copy(dst_hbm.at[wid, ci], sd[b].at[0], ssd[b]).wait()
        pltpu.sync_copy(rows[b], acc_sh.at[sd[b].at[0]], add=True)

        @pl.when(ci + 2 < NCHUNK)
        def _():
          pltpu.async_copy(dst_hbm.at[wid, ci + 2], sd[b].at[0], ssd[b])
          pltpu.async_copy(x_hbm.at[src_v.at[ci + 2]], rows[b], ssa[b])

    plsc.subcore_barrier()

    row0 = pl.multiple_of(s * SLAB, 8)

    @pl.when(s < NS - 1)
    def _():
      pltpu.sync_copy(acc_sh.at[pl.ds(row0, SLAB)],
                      out_hbm.at[c, pl.ds(row0, SLAB)])

    last = N_NODES - (NS - 1) * SLAB

    @pl.when(s == NS - 1)
    def _():
      pltpu.sync_copy(acc_sh.at[pl.ds((NS - 1) * SLAB, last)],
                      out_hbm.at[c, pl.ds((NS - 1) * SLAB, last)])

  return agg(x, src3, dst3)


BLK = 2000


def _mlp(partials, W1, b1, W2, b2):
  def body(p_ref, w1_ref, b1_ref, w2_ref, b2_ref, o_ref):
    ax = p_ref[0] + p_ref[1]
    h = jnp.dot(ax.astype(jnp.bfloat16), w1_ref[...].astype(jnp.bfloat16),
                preferred_element_type=jnp.float32)
    h = jnp.maximum(h + b1_ref[...], 0.0)
    o_ref[...] = (jnp.dot(h.astype(jnp.bfloat16),
                          w2_ref[...].astype(jnp.bfloat16),
                          preferred_element_type=jnp.float32) + b2_ref[...])

  return pl.pallas_call(
      body,
      grid=(N_NODES // BLK,),
      in_specs=[
          pl.BlockSpec((NC, BLK, D_IN), lambda i: (0, i, 0)),
          pl.BlockSpec((D_IN, D_HID), lambda i: (0, 0)),
          pl.BlockSpec((1, D_HID), lambda i: (0, 0)),
          pl.BlockSpec((D_HID, D_OUT), lambda i: (0, 0)),
          pl.BlockSpec((1, D_OUT), lambda i: (0, 0)),
      ],
      out_specs=pl.BlockSpec((BLK, D_OUT), lambda i: (i, 0)),
      out_shape=jax.ShapeDtypeStruct((N_NODES, D_OUT), jnp.float32),
  )(partials, W1, b1.reshape(1, D_HID), W2, b2.reshape(1, D_OUT))


def kernel(x, edge_index, W1, b1, W2, b2):
  ei = edge_index.astype(jnp.int32)
  npad = E_PAD - N_EDGES
  pad_src = jnp.arange(npad, dtype=jnp.int32) % N_NODES
  src = jnp.concatenate([ei[0], pad_src])
  pad_dst = N_NODES + (jnp.arange(npad, dtype=jnp.int32) % (N_PAD - N_NODES))
  dst = jnp.concatenate([ei[1], pad_dst])
  src3 = src.reshape(NW, NCHUNK, CHUNK)
  dst3 = dst.reshape(NW, NCHUNK, CHUNK)
  partials = _sc_aggregate(x, src3, dst3)
  return _mlp(partials, W1, b1, W2, b2)
